# baseline (speedup 1.0000x reference)
import jax
import jax.numpy as jnp
from jax.experimental import pallas as pl
from jax.experimental.pallas import tpu as pltpu

ROWS = 128
COLS = 32768
CHUNKS = (8, 16, 24, 32, 32, 8, 8)
assert sum(CHUNKS) == ROWS
OFFS = [sum(CHUNKS[:i]) for i in range(len(CHUNKS))]
PRIME = 3


def _body(in_ref, out_ref, *scratch):
    n = len(CHUNKS)
    bufs = list(scratch[:n])
    sems = scratch[n]

    def copy(k):
        return pltpu.make_async_copy(
            in_ref.at[pl.ds(OFFS[k], CHUNKS[k])], bufs[k], sems.at[k]
        )

    for k in range(PRIME):
        copy(k).start()

    idxs = []
    for k in range(n):
        if k + PRIME < n:
            copy(k + PRIME).start()
        copy(k).wait()
        a = jnp.argmax(bufs[k][...], axis=1)
        idxs.append(a.reshape(CHUNKS[k], 1).astype(jnp.float32))

    idx_f = jnp.concatenate(idxs, axis=0)
    out_ref[...] = jnp.transpose(idx_f).astype(jnp.int32)


def kernel(inputs):
    out = pl.pallas_call(
        _body,
        in_specs=[pl.BlockSpec(memory_space=pl.ANY)],
        out_specs=pl.BlockSpec(memory_space=pltpu.VMEM),
        out_shape=jax.ShapeDtypeStruct((1, ROWS), jnp.int32),
        scratch_shapes=[pltpu.VMEM((rb, COLS), jnp.float32) for rb in CHUNKS]
        + [pltpu.SemaphoreType.DMA((len(CHUNKS),))],
    )(inputs)
    return out.reshape(ROWS)
